# Initial kernel scaffold; baseline (speedup 1.0000x reference)
import jax
import jax.numpy as jnp
from jax.experimental import pallas as pl


def kernel(x, edge_index, W1, b1, W2, b2):
    raise NotImplementedError("write your pallas kernel here")


# speedup vs baseline: 1.0095x; 1.0095x over previous
import functools

import jax
import jax.numpy as jnp
from jax import lax
from jax.experimental import pallas as pl
from jax.experimental.pallas import tpu as pltpu
from jax.experimental.pallas import tpu_sc as plsc

N_NODES = 10000
D_FEAT = 128
N_EDGES = 320000
HID = 64

NC = 2
NS = 16
NW = NC * NS

_BN = 2000


def _proj_body(x_ref, w_ref, b_ref, pq_ref):
    pq_ref[...] = (
        jnp.dot(x_ref[...], w_ref[...], preferred_element_type=jnp.float32)
        + b_ref[...]
    )


def _project(x, wc, bc):
    return pl.pallas_call(
        _proj_body,
        grid=(N_NODES // _BN,),
        in_specs=[
            pl.BlockSpec((_BN, D_FEAT), lambda i: (i, 0)),
            pl.BlockSpec((D_FEAT, 2 * HID), lambda i: (0, 0)),
            pl.BlockSpec((1, 2 * HID), lambda i: (0, 0)),
        ],
        out_specs=pl.BlockSpec((_BN, 2 * HID), lambda i: (i, 0)),
        out_shape=jax.ShapeDtypeStruct((N_NODES, 2 * HID), jnp.float32),
    )(x, wc, bc)


_EPW = N_EDGES // NW
_GW = 80
_NWIN = _EPW // _GW


def _gather_body(pq_hbm, dst_hbm, src_hbm, g_hbm, idxd, idxs, bufd, bufs, bufg,
                 sem1, sem2):
    wid = lax.axis_index("s") * NC + lax.axis_index("c")
    base = wid * _EPW

    def win(w, carry):
        eb = base + w * _GW
        pltpu.sync_copy(dst_hbm.at[pl.ds(eb, _GW)], idxd)
        pltpu.sync_copy(src_hbm.at[pl.ds(eb, _GW)], idxs)
        cp1 = pltpu.async_copy(pq_hbm.at[idxd], bufd, sem1)
        cp2 = pltpu.async_copy(pq_hbm.at[idxs], bufs, sem2)
        cp1.wait()
        cp2.wait()

        def comp(i, c2):
            e = i // 4
            k = (i % 4) * 16
            v = jnp.maximum(
                bufd[e, pl.ds(k, 16)] + bufs[e, pl.ds(HID + k, 16)], 0.0
            )
            bufg[e, pl.ds(k, 16)] = v
            return c2

        lax.fori_loop(0, _GW * 4, comp, 0)
        pltpu.sync_copy(bufg, g_hbm.at[pl.ds(eb, _GW)])
        return carry

    lax.fori_loop(0, _NWIN, win, 0)


def _sc_gather(pq, dst, src):
    f = functools.partial(
        pl.kernel,
        out_type=jax.ShapeDtypeStruct((N_EDGES, HID), jnp.float32),
        mesh=plsc.VectorSubcoreMesh(core_axis_name="c", subcore_axis_name="s"),
        compiler_params=pltpu.CompilerParams(needs_layout_passes=False),
        scratch_types=[
            pltpu.VMEM((_GW,), jnp.int32),
            pltpu.VMEM((_GW,), jnp.int32),
            pltpu.VMEM((_GW, 2 * HID), jnp.float32),
            pltpu.VMEM((_GW, 2 * HID), jnp.float32),
            pltpu.VMEM((_GW, HID), jnp.float32),
            pltpu.SemaphoreType.DMA,
            pltpu.SemaphoreType.DMA,
        ],
    )(_gather_body)
    return f(pq, dst, src)


_BE = 8000


def _mlp2_body(g_ref, w2_ref, b2_ref, h_ref):
    h_ref[...] = (
        jnp.dot(g_ref[...], w2_ref[...], preferred_element_type=jnp.float32)
        + b2_ref[...]
    )


def _mlp2(g, w2, b2):
    return pl.pallas_call(
        _mlp2_body,
        grid=(N_EDGES // _BE,),
        in_specs=[
            pl.BlockSpec((_BE, HID), lambda i: (i, 0)),
            pl.BlockSpec((HID, D_FEAT), lambda i: (0, 0)),
            pl.BlockSpec((1, D_FEAT), lambda i: (0, 0)),
        ],
        out_specs=pl.BlockSpec((_BE, D_FEAT), lambda i: (i, 0)),
        out_shape=jax.ShapeDtypeStruct((N_EDGES, D_FEAT), jnp.float32),
    )(g, w2, b2)


_DP = 20
_DNR = N_NODES // _DP
_DACC = 512
_DW = 2000
_DNWIN = _EPW // _DW
_GCH = 128
_NQ = 4
_HF = _DACC * D_FEAT // _NQ
_STR = _DACC // _NQ // NS
_FL = _STR * D_FEAT


def _smax_body(h_hbm, dst_hbm, part_hbm, dwin, idlist, dlist, hbuf, acc, rbuf,
               tbuf, spmem, semg):
    c = lax.axis_index("c")
    s = lax.axis_index("s")
    wid = s * NC + c
    ebase = wid * _EPW
    iota = lax.iota(jnp.int32, 16)
    neg = jnp.full((16,), -jnp.inf, jnp.float32)
    pad_ids = jnp.full((16,), 0, jnp.int32) + ebase
    pad_rows = jnp.full((16,), _DNR, jnp.int32)

    def do_pass(p, carry):
        nbase = p * _DNR

        def initf(i, c2):
            acc[pl.ds(i * 16, 16)] = neg
            return c2

        lax.fori_loop(0, _DACC * D_FEAT // 16, initf, 0)

        def win(w, c2):
            ebw = ebase + w * _DW
            pltpu.sync_copy(dst_hbm.at[pl.ds(ebw, _DW)], dwin)

            def filt(i, ptr):
                dv = dwin[pl.ds(i * 16, 16)]
                du = dv - nbase
                m = (du >= 0) & (du < _DNR)
                ids = ebw + i * 16 + iota
                mi = m.astype(jnp.int32)
                pos = ptr + plsc.cumsum(mi) - 1
                plsc.store_scatter(idlist, [pos], ids, mask=m)
                plsc.store_scatter(dlist, [pos], du, mask=m)
                return ptr + jnp.sum(mi)

            cnt = lax.fori_loop(0, _DW // 16, filt, 0)

            def padf(j, c3):
                idlist[pl.ds(cnt + j * 16, 16)] = pad_ids
                dlist[pl.ds(cnt + j * 16, 16)] = pad_rows
                return c3

            lax.fori_loop(0, _GCH // 16, padf, 0)
            nch = (cnt + _GCH - 1) // _GCH

            def drain(k, c3):
                idx = idlist.at[pl.ds(k * _GCH, _GCH)]
                pltpu.async_copy(h_hbm.at[idx], hbuf, semg).wait()

                def rmw(i, c4):
                    rv = dlist[pl.ds(k * _GCH + i, 16)]
                    off = rv[0] * D_FEAT
                    for kk in range(D_FEAT // 16):
                        sl = pl.ds(off + kk * 16, 16)
                        acc[sl] = jnp.maximum(acc[sl], hbuf[i, pl.ds(kk * 16, 16)])
                    return c4

                lax.fori_loop(0, _GCH, rmw, 0)
                return c3

            lax.fori_loop(0, nch, drain, 0)
            return c2

        lax.fori_loop(0, _DNWIN, win, 0)

        def half(hh, c1):
            pltpu.sync_copy(acc.at[pl.ds(hh * _HF, _HF)], spmem.at[s])
            plsc.subcore_barrier()
            pltpu.sync_copy(spmem.at[0, pl.ds(s * _FL, _FL)], rbuf)

            def tloop(t, c2):
                pltpu.sync_copy(spmem.at[t, pl.ds(s * _FL, _FL)], tbuf)

                def vm(i, c3):
                    sl = pl.ds(i * 16, 16)
                    rbuf[sl] = jnp.maximum(rbuf[sl], tbuf[sl])
                    return c3

                lax.fori_loop(0, _FL // 16, vm, 0)
                return c2

            lax.fori_loop(1, NS, tloop, 0)
            pltpu.sync_copy(rbuf, part_hbm.at[c, p, pl.ds(hh * _HF + s * _FL, _FL)])
            plsc.subcore_barrier()
            return c1

        lax.fori_loop(0, _NQ, half, 0)
        return carry

    lax.fori_loop(0, _DP, do_pass, 0)


def _sc_segmax(h, dst):
    f = functools.partial(
        pl.kernel,
        out_type=jax.ShapeDtypeStruct((NC, _DP, _DACC * D_FEAT), jnp.float32),
        mesh=plsc.VectorSubcoreMesh(core_axis_name="c", subcore_axis_name="s"),
        compiler_params=pltpu.CompilerParams(needs_layout_passes=False),
        scratch_types=[
            pltpu.VMEM((_DW,), jnp.int32),
            pltpu.VMEM((_DW + _GCH,), jnp.int32),
            pltpu.VMEM((_DW + _GCH,), jnp.int32),
            pltpu.VMEM((_GCH, D_FEAT), jnp.float32),
            pltpu.VMEM((_DACC * D_FEAT,), jnp.float32),
            pltpu.VMEM((_FL,), jnp.float32),
            pltpu.VMEM((_FL,), jnp.float32),
            pltpu.VMEM_SHARED((NS, _HF), jnp.float32),
            pltpu.SemaphoreType.DMA,
        ],
    )(_smax_body)
    return f(h, dst)


def _comb_body(part_ref, out_ref):
    m = jnp.max(part_ref[...], axis=0)
    m = m[:, :_DNR, :].reshape(N_NODES, D_FEAT)
    out_ref[...] = jnp.where(jnp.isfinite(m), m, 0.0)


def _combine(part):
    return pl.pallas_call(
        _comb_body,
        out_shape=jax.ShapeDtypeStruct((N_NODES, D_FEAT), jnp.float32),
    )(part)


def kernel(x, edge_index, W1, b1, W2, b2):
    src = edge_index[0].astype(jnp.int32)
    dst = edge_index[1].astype(jnp.int32)
    wc = jnp.concatenate([W1[:D_FEAT] - W1[D_FEAT:], W1[D_FEAT:]], axis=1)
    bc = jnp.concatenate([b1, jnp.zeros((HID,), jnp.float32)])
    pq = _project(x, wc, bc[None, :])
    g = _sc_gather(pq, dst, src)
    h = _mlp2(g, W2, b2[None, :])
    part = _sc_segmax(h, dst)
    part = part.reshape(NC, _DP, _DACC, D_FEAT)
    return _combine(part)


# speedup vs baseline: 1.1275x; 1.1169x over previous
import functools

import jax
import jax.numpy as jnp
from jax import lax
from jax.experimental import pallas as pl
from jax.experimental.pallas import tpu as pltpu
from jax.experimental.pallas import tpu_sc as plsc

N_NODES = 10000
D_FEAT = 128
N_EDGES = 320000
HID = 64

NC = 2
NS = 16
NW = NC * NS

_BN = 2000


def _proj_body(x_ref, w_ref, b_ref, pq_ref):
    pq_ref[...] = (
        jnp.dot(x_ref[...], w_ref[...], preferred_element_type=jnp.float32)
        + b_ref[...]
    )


def _project(x, wc, bc):
    return pl.pallas_call(
        _proj_body,
        grid=(N_NODES // _BN,),
        in_specs=[
            pl.BlockSpec((_BN, D_FEAT), lambda i: (i, 0)),
            pl.BlockSpec((D_FEAT, 2 * HID), lambda i: (0, 0)),
            pl.BlockSpec((1, 2 * HID), lambda i: (0, 0)),
        ],
        out_specs=pl.BlockSpec((_BN, 2 * HID), lambda i: (i, 0)),
        out_shape=jax.ShapeDtypeStruct((N_NODES, 2 * HID), jnp.float32),
    )(x, wc, bc)


_EPW = N_EDGES // NW
_GW = 80
_NWIN = _EPW // _GW


def _gather_body(pq_hbm, dst_hbm, src_hbm, g_hbm, idxd, idxs, bufd, bufs, bufg,
                 semd, sems, semo):
    wid = lax.axis_index("s") * NC + lax.axis_index("c")
    base = wid * _EPW
    pltpu.sync_copy(dst_hbm.at[pl.ds(base, _EPW)], idxd)
    pltpu.sync_copy(src_hbm.at[pl.ds(base, _EPW)], idxs)

    def issue(w, slot):
        pltpu.async_copy(
            pq_hbm.at[idxd.at[pl.ds(w * _GW, _GW)]], bufd.at[slot], semd)
        pltpu.async_copy(
            pq_hbm.at[idxs.at[pl.ds(w * _GW, _GW)]], bufs.at[slot], sems)

    def process(w, slot, drain_out):
        eb = base + w * _GW
        pltpu.make_async_copy(
            pq_hbm.at[idxd.at[pl.ds(w * _GW, _GW)]], bufd.at[slot], semd).wait()
        pltpu.make_async_copy(
            pq_hbm.at[idxs.at[pl.ds(w * _GW, _GW)]], bufs.at[slot], sems).wait()
        if drain_out:
            pltpu.make_async_copy(
                g_hbm.at[pl.ds(eb, _GW)], bufg.at[slot], semo).wait()

        def comp(i, c2):
            e = i // 4
            k = (i % 4) * 16
            v = jnp.maximum(
                bufd[slot, e, pl.ds(k, 16)] + bufs[slot, e, pl.ds(HID + k, 16)],
                0.0,
            )
            bufg[slot, e, pl.ds(k, 16)] = v
            return c2

        lax.fori_loop(0, _GW * 4, comp, 0)
        pltpu.async_copy(bufg.at[slot], g_hbm.at[pl.ds(eb, _GW)], semo)

    issue(0, 0)

    def pair(ii, carry):
        issue(2 * ii + 1, 1)
        process(2 * ii, 0, drain_out=True)
        issue(2 * ii + 2, 0)
        process(2 * ii + 1, 1, drain_out=True)
        return carry

    issue(1, 1)
    process(0, 0, drain_out=False)
    issue(2, 0)
    process(1, 1, drain_out=False)
    lax.fori_loop(1, (_NWIN - 1) // 2, pair, 0)
    process(_NWIN - 1, 0, drain_out=True)
    pltpu.make_async_copy(g_hbm.at[pl.ds(base, _GW)], bufg.at[0], semo).wait()
    pltpu.make_async_copy(g_hbm.at[pl.ds(base, _GW)], bufg.at[1], semo).wait()


def _sc_gather(pq, dst, src):
    f = functools.partial(
        pl.kernel,
        out_type=jax.ShapeDtypeStruct((N_EDGES, HID), jnp.float32),
        mesh=plsc.VectorSubcoreMesh(core_axis_name="c", subcore_axis_name="s"),
        compiler_params=pltpu.CompilerParams(needs_layout_passes=False),
        scratch_types=[
            pltpu.VMEM((_EPW,), jnp.int32),
            pltpu.VMEM((_EPW,), jnp.int32),
            pltpu.VMEM((2, _GW, 2 * HID), jnp.float32),
            pltpu.VMEM((2, _GW, 2 * HID), jnp.float32),
            pltpu.VMEM((2, _GW, HID), jnp.float32),
            pltpu.SemaphoreType.DMA,
            pltpu.SemaphoreType.DMA,
            pltpu.SemaphoreType.DMA,
        ],
    )(_gather_body)
    return f(pq, dst, src)


_BE = 8000


def _mlp2_body(g_ref, w2_ref, b2_ref, h_ref):
    h_ref[...] = (
        jnp.dot(g_ref[...], w2_ref[...], preferred_element_type=jnp.float32)
        + b2_ref[...]
    )


def _mlp2(g, w2, b2):
    return pl.pallas_call(
        _mlp2_body,
        grid=(N_EDGES // _BE,),
        in_specs=[
            pl.BlockSpec((_BE, HID), lambda i: (i, 0)),
            pl.BlockSpec((HID, D_FEAT), lambda i: (0, 0)),
            pl.BlockSpec((1, D_FEAT), lambda i: (0, 0)),
        ],
        out_specs=pl.BlockSpec((_BE, D_FEAT), lambda i: (i, 0)),
        out_shape=jax.ShapeDtypeStruct((N_EDGES, D_FEAT), jnp.float32),
    )(g, w2, b2)


_DP = 25
_DNR = N_NODES // _DP
_DACC = 416
_DW = 2000
_DNWIN = _EPW // _DW
_GCH = 128


def _smax_body(h_hbm, dst_hbm, part_hbm, dwin, idlist, dlist, hbuf, acc, semg):
    c = lax.axis_index("c")
    s = lax.axis_index("s")
    wid = s * NC + c
    ebase = wid * _EPW
    iota = lax.iota(jnp.int32, 16)
    neg = jnp.full((16,), -jnp.inf, jnp.float32)
    pad_ids = jnp.full((16,), 0, jnp.int32) + ebase
    pad_rows = jnp.full((16,), _DNR, jnp.int32)

    pltpu.sync_copy(dst_hbm.at[pl.ds(ebase, _EPW)], dwin)

    def do_pass(p, carry):
        nbase = p * _DNR

        def initf(i, c2):
            acc[pl.ds(i * 16, 16)] = neg
            return c2

        lax.fori_loop(0, _DACC * D_FEAT // 16, initf, 0)

        def win(w, c2):
            ebw = ebase + w * _DW

            def filt(i, ptr):
                dv = dwin[pl.ds(w * _DW + i * 16, 16)]
                du = dv - nbase
                m = (du >= 0) & (du < _DNR)
                ids = ebw + i * 16 + iota
                mi = m.astype(jnp.int32)
                pos = ptr + plsc.cumsum(mi) - 1
                plsc.store_scatter(idlist, [pos], ids, mask=m)
                plsc.store_scatter(dlist, [pos], du, mask=m)
                return ptr + jnp.sum(mi)

            cnt = lax.fori_loop(0, _DW // 16, filt, 0)

            def padf(j, c3):
                idlist[pl.ds(cnt + j * 16, 16)] = pad_ids
                dlist[pl.ds(cnt + j * 16, 16)] = pad_rows
                return c3

            lax.fori_loop(0, _GCH // 16, padf, 0)
            nch = (cnt + _GCH - 1) // _GCH

            def drain(k, c3):
                idx = idlist.at[pl.ds(k * _GCH, _GCH)]
                pltpu.async_copy(h_hbm.at[idx], hbuf, semg).wait()

                def rmw(i, c4):
                    rv = dlist[pl.ds(k * _GCH + i, 16)]
                    off = rv[0] * D_FEAT
                    for kk in range(D_FEAT // 16):
                        sl = pl.ds(off + kk * 16, 16)
                        acc[sl] = jnp.maximum(acc[sl], hbuf[i, pl.ds(kk * 16, 16)])
                    return c4

                lax.fori_loop(0, _GCH, rmw, 0)
                return c3

            lax.fori_loop(0, nch, drain, 0)
            return c2

        lax.fori_loop(0, _DNWIN, win, 0)
        pltpu.sync_copy(acc, part_hbm.at[c, p, s])
        return carry

    lax.fori_loop(0, _DP, do_pass, 0)


def _sc_segmax(h, dst):
    f = functools.partial(
        pl.kernel,
        out_type=jax.ShapeDtypeStruct((NC, _DP, NS, _DACC * D_FEAT), jnp.float32),
        mesh=plsc.VectorSubcoreMesh(core_axis_name="c", subcore_axis_name="s"),
        compiler_params=pltpu.CompilerParams(needs_layout_passes=False),
        scratch_types=[
            pltpu.VMEM((_EPW,), jnp.int32),
            pltpu.VMEM((_DW + _GCH,), jnp.int32),
            pltpu.VMEM((_DW + _GCH,), jnp.int32),
            pltpu.VMEM((_GCH, D_FEAT), jnp.float32),
            pltpu.VMEM((_DACC * D_FEAT,), jnp.float32),
            pltpu.SemaphoreType.DMA,
        ],
    )(_smax_body)
    return f(h, dst)


def _comb_body(part_ref, out_ref):
    m = jnp.max(part_ref[...], axis=(0, 1, 2))
    m = m[:_DNR]
    out_ref[...] = jnp.where(jnp.isfinite(m), m, 0.0)


def _combine(part):
    return pl.pallas_call(
        _comb_body,
        grid=(_DP,),
        in_specs=[
            pl.BlockSpec((NC, 1, NS, _DACC, D_FEAT), lambda p: (0, p, 0, 0, 0)),
        ],
        out_specs=pl.BlockSpec((_DNR, D_FEAT), lambda p: (p, 0)),
        out_shape=jax.ShapeDtypeStruct((N_NODES, D_FEAT), jnp.float32),
    )(part)


def kernel(x, edge_index, W1, b1, W2, b2):
    src = edge_index[0].astype(jnp.int32)
    dst = edge_index[1].astype(jnp.int32)
    wc = jnp.concatenate([W1[:D_FEAT] - W1[D_FEAT:], W1[D_FEAT:]], axis=1)
    bc = jnp.concatenate([b1, jnp.zeros((HID,), jnp.float32)])
    pq = _project(x, wc, bc[None, :])
    g = _sc_gather(pq, dst, src)
    h = _mlp2(g, W2, b2[None, :])
    part = _sc_segmax(h, dst)
    part = part.reshape(NC, _DP, NS, _DACC, D_FEAT)
    return _combine(part)


# speedup vs baseline: 1.8576x; 1.6476x over previous
import functools

import jax
import jax.numpy as jnp
from jax import lax
from jax.experimental import pallas as pl
from jax.experimental.pallas import tpu as pltpu
from jax.experimental.pallas import tpu_sc as plsc

N_NODES = 10000
D_FEAT = 128
N_EDGES = 320000
HID = 64

NC = 2
NS = 16
NW = NC * NS

_BN = 2000


def _proj_body(x_ref, w_ref, b_ref, pq_ref):
    pq_ref[...] = (
        jnp.dot(x_ref[...], w_ref[...], preferred_element_type=jnp.float32)
        + b_ref[...]
    )


def _project(x, wc, bc):
    return pl.pallas_call(
        _proj_body,
        grid=(N_NODES // _BN,),
        in_specs=[
            pl.BlockSpec((_BN, D_FEAT), lambda i: (i, 0)),
            pl.BlockSpec((D_FEAT, 2 * HID), lambda i: (0, 0)),
            pl.BlockSpec((1, 2 * HID), lambda i: (0, 0)),
        ],
        out_specs=pl.BlockSpec((_BN, 2 * HID), lambda i: (i, 0)),
        out_shape=jax.ShapeDtypeStruct((N_NODES, 2 * HID), jnp.float32),
    )(x, wc, bc)


_EPW = N_EDGES // NW
_GW = 80
_NWIN = _EPW // _GW


def _gather_body(pq_hbm, dst_hbm, src_hbm, g_hbm, idxd, idxs, bufd, bufs, bufg,
                 semd, sems, semo):
    wid = lax.axis_index("s") * NC + lax.axis_index("c")
    base = wid * _EPW
    pltpu.sync_copy(dst_hbm.at[pl.ds(base, _EPW)], idxd)
    pltpu.sync_copy(src_hbm.at[pl.ds(base, _EPW)], idxs)

    def issue(w, slot):
        pltpu.async_copy(
            pq_hbm.at[idxd.at[pl.ds(w * _GW, _GW)]], bufd.at[slot], semd)
        pltpu.async_copy(
            pq_hbm.at[idxs.at[pl.ds(w * _GW, _GW)]], bufs.at[slot], sems)

    def process(w, slot, drain_out):
        eb = base + w * _GW
        pltpu.make_async_copy(
            pq_hbm.at[idxd.at[pl.ds(w * _GW, _GW)]], bufd.at[slot], semd).wait()
        pltpu.make_async_copy(
            pq_hbm.at[idxs.at[pl.ds(w * _GW, _GW)]], bufs.at[slot], sems).wait()
        if drain_out:
            pltpu.make_async_copy(
                g_hbm.at[pl.ds(eb, _GW)], bufg.at[slot], semo).wait()

        def comp(i, c2):
            e = i // 4
            k = (i % 4) * 16
            v = jnp.maximum(
                bufd[slot, e, pl.ds(k, 16)] + bufs[slot, e, pl.ds(HID + k, 16)],
                0.0,
            )
            bufg[slot, e, pl.ds(k, 16)] = v
            return c2

        lax.fori_loop(0, _GW * 4, comp, 0, unroll=4)
        pltpu.async_copy(bufg.at[slot], g_hbm.at[pl.ds(eb, _GW)], semo)

    issue(0, 0)

    def pair(ii, carry):
        issue(2 * ii + 1, 1)
        process(2 * ii, 0, drain_out=True)
        issue(2 * ii + 2, 0)
        process(2 * ii + 1, 1, drain_out=True)
        return carry

    issue(1, 1)
    process(0, 0, drain_out=False)
    issue(2, 0)
    process(1, 1, drain_out=False)
    lax.fori_loop(1, (_NWIN - 1) // 2, pair, 0)
    process(_NWIN - 1, 0, drain_out=True)
    pltpu.make_async_copy(g_hbm.at[pl.ds(base, _GW)], bufg.at[0], semo).wait()
    pltpu.make_async_copy(g_hbm.at[pl.ds(base, _GW)], bufg.at[1], semo).wait()


def _sc_gather(pq, dst, src):
    f = functools.partial(
        pl.kernel,
        out_type=jax.ShapeDtypeStruct((N_EDGES, HID), jnp.float32),
        mesh=plsc.VectorSubcoreMesh(core_axis_name="c", subcore_axis_name="s"),
        compiler_params=pltpu.CompilerParams(needs_layout_passes=False),
        scratch_types=[
            pltpu.VMEM((_EPW,), jnp.int32),
            pltpu.VMEM((_EPW,), jnp.int32),
            pltpu.VMEM((2, _GW, 2 * HID), jnp.float32),
            pltpu.VMEM((2, _GW, 2 * HID), jnp.float32),
            pltpu.VMEM((2, _GW, HID), jnp.float32),
            pltpu.SemaphoreType.DMA,
            pltpu.SemaphoreType.DMA,
            pltpu.SemaphoreType.DMA,
        ],
    )(_gather_body)
    return f(pq, dst, src)


_BE = 8000


def _mlp2_body(g_ref, w2_ref, b2_ref, h_ref):
    h_ref[...] = (
        jnp.dot(g_ref[...], w2_ref[...], preferred_element_type=jnp.float32)
        + b2_ref[...]
    )


def _mlp2(g, w2, b2):
    return pl.pallas_call(
        _mlp2_body,
        grid=(N_EDGES // _BE,),
        in_specs=[
            pl.BlockSpec((_BE, HID), lambda i: (i, 0)),
            pl.BlockSpec((HID, D_FEAT), lambda i: (0, 0)),
            pl.BlockSpec((1, D_FEAT), lambda i: (0, 0)),
        ],
        out_specs=pl.BlockSpec((_BE, D_FEAT), lambda i: (i, 0)),
        out_shape=jax.ShapeDtypeStruct((N_EDGES, D_FEAT), jnp.float32),
    )(g, w2, b2)


_DP = 25
_DNR = N_NODES // _DP
_DACC = 416
_GCH = 128
_LCAP = _EPW + _GCH


def _smax_body(h_hbm, dst_hbm, part_hbm, dwin, idlist, dlist, hbuf, acc, semg):
    c = lax.axis_index("c")
    s = lax.axis_index("s")
    wid = s * NC + c
    ebase = wid * _EPW
    iota = lax.iota(jnp.int32, 16)
    neg = jnp.full((16,), -jnp.inf, jnp.float32)
    pad_ids = jnp.full((16,), 0, jnp.int32) + ebase
    pad_rows = jnp.full((16,), _DNR, jnp.int32)

    pltpu.sync_copy(dst_hbm.at[pl.ds(ebase, _EPW)], dwin)

    def do_pass(p, carry):
        nbase = p * _DNR

        def initf(i, c2):
            acc[pl.ds(i * 16, 16)] = neg
            return c2

        with jax.named_scope("smax_init"):
            lax.fori_loop(0, _DACC * D_FEAT // 16, initf, 0, unroll=8)

        def filt(i, ptr):
            dv = dwin[pl.ds(i * 16, 16)]
            du = dv - nbase
            m = (du >= 0) & (du < _DNR)
            ids = ebase + i * 16 + iota
            mi = m.astype(jnp.int32)
            pos = ptr + plsc.cumsum(mi) - 1
            plsc.store_scatter(idlist, [pos], ids, mask=m)
            plsc.store_scatter(dlist, [pos], du, mask=m)
            return ptr + jnp.sum(mi)

        with jax.named_scope("smax_filter"):
            cnt = lax.fori_loop(0, _EPW // 16, filt, 0, unroll=4)

        def padf(j, c3):
            idlist[pl.ds(cnt + j * 16, 16)] = pad_ids
            dlist[pl.ds(cnt + j * 16, 16)] = pad_rows
            return c3

        lax.fori_loop(0, _GCH // 16, padf, 0)
        nch = (cnt + _GCH - 1) // _GCH

        def gissue(k, slot):
            pltpu.async_copy(
                h_hbm.at[idlist.at[pl.ds(k * _GCH, _GCH)]], hbuf.at[slot], semg)

        @pl.when(nch > 0)
        def _prime():
            gissue(0, 0)

        def drain(k, c3):
            slot = lax.rem(k, 2)

            @pl.when(k + 1 < nch)
            def _pre():
                gissue(k + 1, 1 - slot)

            pltpu.make_async_copy(
                h_hbm.at[idlist.at[pl.ds(k * _GCH, _GCH)]], hbuf.at[slot],
                semg).wait()

            def rmw(i, c4):
                rv = dlist[pl.ds(k * _GCH + i, 16)]
                off = rv[0] * D_FEAT
                for kk in range(D_FEAT // 16):
                    sl = pl.ds(off + kk * 16, 16)
                    acc[sl] = jnp.maximum(
                        acc[sl], hbuf[slot, i, pl.ds(kk * 16, 16)])
                return c4

            with jax.named_scope("smax_rmw"):
                lax.fori_loop(0, _GCH, rmw, 0, unroll=4)
            return c3

        with jax.named_scope("smax_drain"):
            lax.fori_loop(0, nch, drain, 0)
        with jax.named_scope("smax_partout"):
            pltpu.sync_copy(acc, part_hbm.at[c, p, s])
        return carry

    lax.fori_loop(0, _DP, do_pass, 0)


def _sc_segmax(h, dst):
    f = functools.partial(
        pl.kernel,
        out_type=jax.ShapeDtypeStruct((NC, _DP, NS, _DACC * D_FEAT), jnp.float32),
        mesh=plsc.VectorSubcoreMesh(core_axis_name="c", subcore_axis_name="s"),
        compiler_params=pltpu.CompilerParams(needs_layout_passes=False),
        scratch_types=[
            pltpu.VMEM((_EPW,), jnp.int32),
            pltpu.VMEM((_LCAP,), jnp.int32),
            pltpu.VMEM((_LCAP,), jnp.int32),
            pltpu.VMEM((2, _GCH, D_FEAT), jnp.float32),
            pltpu.VMEM((_DACC * D_FEAT,), jnp.float32),
            pltpu.SemaphoreType.DMA,
        ],
    )(_smax_body)
    return f(h, dst)


def _comb_body(part_ref, out_ref):
    m = jnp.max(part_ref[...], axis=(0, 1, 2))
    m = m[:_DNR]
    out_ref[...] = jnp.where(jnp.isfinite(m), m, 0.0)


def _combine(part):
    return pl.pallas_call(
        _comb_body,
        grid=(_DP,),
        in_specs=[
            pl.BlockSpec((NC, 1, NS, _DACC, D_FEAT), lambda p: (0, p, 0, 0, 0)),
        ],
        out_specs=pl.BlockSpec((_DNR, D_FEAT), lambda p: (p, 0)),
        out_shape=jax.ShapeDtypeStruct((N_NODES, D_FEAT), jnp.float32),
    )(part)


def kernel(x, edge_index, W1, b1, W2, b2):
    src = edge_index[0].astype(jnp.int32)
    dst = edge_index[1].astype(jnp.int32)
    wc = jnp.concatenate([W1[:D_FEAT] - W1[D_FEAT:], W1[D_FEAT:]], axis=1)
    bc = jnp.concatenate([b1, jnp.zeros((HID,), jnp.float32)])
    pq = _project(x, wc, bc[None, :])
    g = _sc_gather(pq, dst, src)
    h = _mlp2(g, W2, b2[None, :])
    part = _sc_segmax(h, dst)
    part = part.reshape(NC, _DP, NS, _DACC, D_FEAT)
    return _combine(part)


# speedup vs baseline: 2.0364x; 1.0962x over previous
import functools

import jax
import jax.numpy as jnp
from jax import lax
from jax.experimental import pallas as pl
from jax.experimental.pallas import tpu as pltpu
from jax.experimental.pallas import tpu_sc as plsc

N_NODES = 10000
D_FEAT = 128
N_EDGES = 320000
HID = 64

NC = 2
NS = 16
NW = NC * NS

_BN = 2000


def _proj_body(x_ref, w_ref, b_ref, pq_ref):
    pq_ref[...] = (
        jnp.dot(x_ref[...], w_ref[...], preferred_element_type=jnp.float32)
        + b_ref[...]
    )


def _project(x, wc, bc):
    return pl.pallas_call(
        _proj_body,
        grid=(N_NODES // _BN,),
        in_specs=[
            pl.BlockSpec((_BN, D_FEAT), lambda i: (i, 0)),
            pl.BlockSpec((D_FEAT, 2 * HID), lambda i: (0, 0)),
            pl.BlockSpec((1, 2 * HID), lambda i: (0, 0)),
        ],
        out_specs=pl.BlockSpec((_BN, 2 * HID), lambda i: (i, 0)),
        out_shape=jax.ShapeDtypeStruct((N_NODES, 2 * HID), jnp.float32),
    )(x, wc, bc)


_EPW = N_EDGES // NW
_GW = 80
_NWIN = _EPW // _GW


def _gather_body(pq_hbm, dst_hbm, src_hbm, g_hbm, idxd, idxs, bufd, bufs, bufg,
                 semd, sems, semo):
    wid = lax.axis_index("s") * NC + lax.axis_index("c")
    base = wid * _EPW
    pltpu.sync_copy(dst_hbm.at[pl.ds(base, _EPW)], idxd)
    pltpu.sync_copy(src_hbm.at[pl.ds(base, _EPW)], idxs)

    def issue(w, slot):
        pltpu.async_copy(
            pq_hbm.at[idxd.at[pl.ds(w * _GW, _GW)]], bufd.at[slot], semd)
        pltpu.async_copy(
            pq_hbm.at[idxs.at[pl.ds(w * _GW, _GW)]], bufs.at[slot], sems)

    def process(w, slot, drain_out):
        eb = base + w * _GW
        pltpu.make_async_copy(
            pq_hbm.at[idxd.at[pl.ds(w * _GW, _GW)]], bufd.at[slot], semd).wait()
        pltpu.make_async_copy(
            pq_hbm.at[idxs.at[pl.ds(w * _GW, _GW)]], bufs.at[slot], sems).wait()
        if drain_out:
            pltpu.make_async_copy(
                g_hbm.at[pl.ds(eb, _GW)], bufg.at[slot], semo).wait()

        def comp(i, c2):
            e = i // 4
            k = (i % 4) * 16
            v = jnp.maximum(
                bufd[slot, e, pl.ds(k, 16)] + bufs[slot, e, pl.ds(HID + k, 16)],
                0.0,
            )
            bufg[slot, e, pl.ds(k, 16)] = v
            return c2

        lax.fori_loop(0, _GW * 4, comp, 0, unroll=4)
        pltpu.async_copy(bufg.at[slot], g_hbm.at[pl.ds(eb, _GW)], semo)

    issue(0, 0)

    def pair(ii, carry):
        issue(2 * ii + 1, 1)
        process(2 * ii, 0, drain_out=True)
        issue(2 * ii + 2, 0)
        process(2 * ii + 1, 1, drain_out=True)
        return carry

    issue(1, 1)
    process(0, 0, drain_out=False)
    issue(2, 0)
    process(1, 1, drain_out=False)
    lax.fori_loop(1, (_NWIN - 1) // 2, pair, 0)
    process(_NWIN - 1, 0, drain_out=True)
    pltpu.make_async_copy(g_hbm.at[pl.ds(base, _GW)], bufg.at[0], semo).wait()
    pltpu.make_async_copy(g_hbm.at[pl.ds(base, _GW)], bufg.at[1], semo).wait()


def _sc_gather(pq, dst, src):
    f = functools.partial(
        pl.kernel,
        out_type=jax.ShapeDtypeStruct((N_EDGES, HID), jnp.float32),
        mesh=plsc.VectorSubcoreMesh(core_axis_name="c", subcore_axis_name="s"),
        compiler_params=pltpu.CompilerParams(needs_layout_passes=False),
        scratch_types=[
            pltpu.VMEM((_EPW,), jnp.int32),
            pltpu.VMEM((_EPW,), jnp.int32),
            pltpu.VMEM((2, _GW, 2 * HID), jnp.float32),
            pltpu.VMEM((2, _GW, 2 * HID), jnp.float32),
            pltpu.VMEM((2, _GW, HID), jnp.float32),
            pltpu.SemaphoreType.DMA,
            pltpu.SemaphoreType.DMA,
            pltpu.SemaphoreType.DMA,
        ],
    )(_gather_body)
    return f(pq, dst, src)


_BE = 8000


def _mlp2_body(g_ref, w2_ref, b2_ref, h_ref):
    h_ref[...] = (
        jnp.dot(g_ref[...], w2_ref[...], preferred_element_type=jnp.float32)
        + b2_ref[...]
    )


def _mlp2(g, w2, b2):
    return pl.pallas_call(
        _mlp2_body,
        grid=(N_EDGES // _BE,),
        in_specs=[
            pl.BlockSpec((_BE, HID), lambda i: (i, 0)),
            pl.BlockSpec((HID, D_FEAT), lambda i: (0, 0)),
            pl.BlockSpec((1, D_FEAT), lambda i: (0, 0)),
        ],
        out_specs=pl.BlockSpec((_BE, D_FEAT), lambda i: (i, 0)),
        out_shape=jax.ShapeDtypeStruct((N_EDGES, D_FEAT), jnp.float32),
    )(g, w2, b2)


_DP = 25
_DNR = N_NODES // _DP
_DACC = 416
_GCH = 128
_LCAP = _EPW + _GCH


def _smax_body(h_hbm, dst_hbm, part_hbm, dwin, idlist, dlist, hbuf, acc, semg):
    c = lax.axis_index("c")
    s = lax.axis_index("s")
    wid = s * NC + c
    ebase = wid * _EPW
    iota = lax.iota(jnp.int32, 16)
    neg = jnp.full((16,), -jnp.inf, jnp.float32)
    pad_ids = jnp.full((16,), 0, jnp.int32) + ebase
    pad_rows = jnp.full((16,), _DNR, jnp.int32)

    pltpu.sync_copy(dst_hbm.at[pl.ds(ebase, _EPW)], dwin)

    def do_pass(p, carry):
        nbase = p * _DNR

        def initf(i, c2):
            for kk in range(D_FEAT // 16):
                acc[i, pl.ds(kk * 16, 16)] = neg
            return c2

        with jax.named_scope("smax_init"):
            lax.fori_loop(0, _DACC, initf, 0, unroll=2)

        def filt(i, ptr):
            dv = dwin[pl.ds(i * 16, 16)]
            du = dv - nbase
            m = (du >= 0) & (du < _DNR)
            ids = ebase + i * 16 + iota
            mi = m.astype(jnp.int32)
            pos = ptr + plsc.cumsum(mi) - 1
            plsc.store_scatter(idlist, [pos], ids, mask=m)
            plsc.store_scatter(dlist, [pos], du, mask=m)
            return ptr + jnp.sum(mi)

        with jax.named_scope("smax_filter"):
            cnt = lax.fori_loop(0, _EPW // 16, filt, 0, unroll=4)

        def padf(j, c3):
            idlist[pl.ds(cnt + j * 16, 16)] = pad_ids
            dlist[pl.ds(cnt + j * 16, 16)] = pad_rows
            return c3

        lax.fori_loop(0, _GCH // 16, padf, 0)
        nch = (cnt + _GCH - 1) // _GCH

        def gissue(k, slot):
            pltpu.async_copy(
                h_hbm.at[idlist.at[pl.ds(k * _GCH, _GCH)]], hbuf.at[slot], semg)

        @pl.when(nch > 0)
        def _prime():
            gissue(0, 0)

        def drain(k, c3):
            slot = lax.rem(k, 2)

            @pl.when(k + 1 < nch)
            def _pre():
                gissue(k + 1, 1 - slot)

            pltpu.make_async_copy(
                h_hbm.at[idlist.at[pl.ds(k * _GCH, _GCH)]], hbuf.at[slot],
                semg).wait()

            def rmw(i, c4):
                rv = dlist[pl.ds(k * _GCH + i, 16)]
                r = rv[0]
                for kk in range(D_FEAT // 16):
                    sl = pl.ds(kk * 16, 16)
                    acc[r, sl] = jnp.maximum(
                        acc[r, sl], hbuf[slot, i, sl])
                return c4

            with jax.named_scope("smax_rmw"):
                lax.fori_loop(0, _GCH, rmw, 0, unroll=4)
            return c3

        with jax.named_scope("smax_drain"):
            lax.fori_loop(0, nch, drain, 0)
        with jax.named_scope("smax_partout"):
            pltpu.sync_copy(acc, part_hbm.at[c, p, s])
        return carry

    lax.fori_loop(0, _DP, do_pass, 0)


def _sc_segmax(h, dst):
    f = functools.partial(
        pl.kernel,
        out_type=jax.ShapeDtypeStruct((NC, _DP, NS, _DACC, D_FEAT), jnp.float32),
        mesh=plsc.VectorSubcoreMesh(core_axis_name="c", subcore_axis_name="s"),
        compiler_params=pltpu.CompilerParams(needs_layout_passes=False),
        scratch_types=[
            pltpu.VMEM((_EPW,), jnp.int32),
            pltpu.VMEM((_LCAP,), jnp.int32),
            pltpu.VMEM((_LCAP,), jnp.int32),
            pltpu.VMEM((2, _GCH, D_FEAT), jnp.float32),
            pltpu.VMEM((_DACC, D_FEAT), jnp.float32),
            pltpu.SemaphoreType.DMA,
        ],
    )(_smax_body)
    return f(h, dst)


def _comb_body(part_ref, out_ref):
    m = jnp.max(part_ref[...], axis=(0, 1, 2))
    m = m[:_DNR]
    out_ref[...] = jnp.where(jnp.isfinite(m), m, 0.0)


def _combine(part):
    return pl.pallas_call(
        _comb_body,
        grid=(_DP,),
        in_specs=[
            pl.BlockSpec((NC, 1, NS, _DACC, D_FEAT), lambda p: (0, p, 0, 0, 0)),
        ],
        out_specs=pl.BlockSpec((_DNR, D_FEAT), lambda p: (p, 0)),
        out_shape=jax.ShapeDtypeStruct((N_NODES, D_FEAT), jnp.float32),
    )(part)


def kernel(x, edge_index, W1, b1, W2, b2):
    src = edge_index[0].astype(jnp.int32)
    dst = edge_index[1].astype(jnp.int32)
    wc = jnp.concatenate([W1[:D_FEAT] - W1[D_FEAT:], W1[D_FEAT:]], axis=1)
    bc = jnp.concatenate([b1, jnp.zeros((HID,), jnp.float32)])
    pq = _project(x, wc, bc[None, :])
    g = _sc_gather(pq, dst, src)
    h = _mlp2(g, W2, b2[None, :])
    part = _sc_segmax(h, dst)
    return _combine(part)
